# baseline (speedup 1.0000x reference)
import jax
import jax.numpy as jnp
from jax.experimental import pallas as pl


def _body(sub_ref, base_ref, out_ref):
    rolled = jnp.roll(jnp.roll(sub_ref[...], 128, axis=3), 128, axis=2)
    out_ref[...] = rolled + base_ref[...]


def kernel(subimg, base_image):
    batch, chans, h, w = subimg.shape
    bblk = 4
    grid = (batch // bblk,)

    sub_spec = pl.BlockSpec(
        (bblk, chans, h, w),
        lambda b: (b, 0, 0, 0),
    )
    base_spec = pl.BlockSpec(
        (1, chans, h, w),
        lambda b: (0, 0, 0, 0),
    )
    out_spec = pl.BlockSpec(
        (bblk, chans, h, w),
        lambda b: (b, 0, 0, 0),
    )

    return pl.pallas_call(
        _body,
        grid=grid,
        in_specs=[sub_spec, base_spec],
        out_specs=out_spec,
        out_shape=jax.ShapeDtypeStruct((batch, chans, h, w), subimg.dtype),
    )(subimg, base_image)
